# Initial kernel scaffold; baseline (speedup 1.0000x reference)
import jax
import jax.numpy as jnp
from jax.experimental import pallas as pl


def kernel(last_tokens, lidar2img, Hp, Wp, w_view, proj_w, bev_query, so_w, so_b, aw_w, aw_b, vp_w, vp_b, op_w, op_b):
    raise NotImplementedError("write your pallas kernel here")


# speedup vs baseline: 3219.8174x; 3219.8174x over previous
import functools

import jax
import jax.numpy as jnp
from jax import lax
from jax.experimental import pallas as pl
from jax.experimental.pallas import tpu as pltpu
from jax.experimental.pallas import tpu_sc as plsc

V = 6
HP = 32
WP = 32
N = HP * WP
CD = 768
C = 256
NH = 8
NP = 4
HD = C // NH
Q = 2500
QPAD = 2560
QH = 1280
EPS = 1e-6
_PREC = lax.Precision.HIGHEST


def _dot(a, b, dims):
    return lax.dot_general(a, b, (dims, ((), ())),
                           preferred_element_type=jnp.float32,
                           precision=_PREC)


def _val_body(tok_ref, pw_ref, vw_ref, vb_ref, out_ref):
    x = tok_ref[0]
    v1 = _dot(x, pw_ref[...], ((1,), (1,)))
    v2 = _dot(v1, vw_ref[...], ((1,), (1,))) + vb_ref[...]
    for h in range(NH):
        out_ref[h] = v2[:, h * HD:(h + 1) * HD]


def _value_maps(tokens, proj_w, vp_w, vp_b):
    return pl.pallas_call(
        _val_body,
        grid=(V,),
        in_specs=[
            pl.BlockSpec((1, N, CD), lambda v: (v, 0, 0)),
            pl.BlockSpec((C, CD), lambda v: (0, 0)),
            pl.BlockSpec((C, C), lambda v: (0, 0)),
            pl.BlockSpec((1, C), lambda v: (0, 0)),
        ],
        out_specs=pl.BlockSpec((NH, N, HD), lambda v: (v, 0, 0)),
        out_shape=jax.ShapeDtypeStruct((V * NH, N, HD), jnp.float32),
    )(tokens, proj_w, vp_w, vp_b.reshape(1, C))


def _idxw_body(q_ref, swx_ref, swy_ref, sbx_ref, sby_ref, aww_ref, awb_ref,
               rx_ref, ry_ref, idx_ref, w_ref):
    qv = q_ref[...]
    offx = _dot(qv, swx_ref[...], ((1,), (1,))) + sbx_ref[...]
    offy = _dot(qv, swy_ref[...], ((1,), (1,))) + sby_ref[...]
    logits = _dot(qv, aww_ref[...], ((1,), (1,))) + awb_ref[...]
    m = jnp.max(logits, axis=1, keepdims=True)
    e = jnp.exp(logits - m)
    gi = lax.broadcasted_iota(jnp.int32, (NH * NP, NH * NP), 0) // NP
    gj = lax.broadcasted_iota(jnp.int32, (NH * NP, NH * NP), 1) // NP
    grp = (gi == gj).astype(jnp.float32)
    s = _dot(e, grp, ((1,), (0,)))
    aw = e / s

    refx = rx_ref[0]
    refy = ry_ref[0]

    px = (refx + offx * (1.0 / WP)) * float(WP) - 0.5
    py = (refy + offy * (1.0 / HP)) * float(HP) - 0.5
    x0 = jnp.floor(px)
    y0 = jnp.floor(py)
    wx1 = px - x0
    wy1 = py - y0
    idxs = []
    ws = []
    for cx, cy in ((0, 0), (1, 0), (0, 1), (1, 1)):
        ix = x0 + cx
        iy = y0 + cy
        vc = ((ix >= 0) & (ix <= WP - 1) & (iy >= 0) & (iy <= HP - 1))
        ixc = jnp.clip(ix, 0.0, WP - 1.0).astype(jnp.int32)
        iyc = jnp.clip(iy, 0.0, HP - 1.0).astype(jnp.int32)
        idxs.append(iyc * WP + ixc)
        wxc = wx1 if cx == 1 else 1.0 - wx1
        wyc = wy1 if cy == 1 else 1.0 - wy1
        ws.append(aw * wxc * wyc * vc.astype(jnp.float32))
    idx_ref[0] = jnp.concatenate(idxs, axis=1)
    w_ref[0] = jnp.concatenate(ws, axis=1)


def _indices_weights(bev_query, so_w, so_b, aw_w, aw_b, refx32, refy32):
    swx = so_w[0::2]
    swy = so_w[1::2]
    sbx = so_b[0::2].reshape(1, NH * NP)
    sby = so_b[1::2].reshape(1, NH * NP)
    return pl.pallas_call(
        _idxw_body,
        grid=(V,),
        in_specs=[
            pl.BlockSpec((Q, C), lambda v: (0, 0)),
            pl.BlockSpec((NH * NP, C), lambda v: (0, 0)),
            pl.BlockSpec((NH * NP, C), lambda v: (0, 0)),
            pl.BlockSpec((1, NH * NP), lambda v: (0, 0)),
            pl.BlockSpec((1, NH * NP), lambda v: (0, 0)),
            pl.BlockSpec((NH * NP, C), lambda v: (0, 0)),
            pl.BlockSpec((1, NH * NP), lambda v: (0, 0)),
            pl.BlockSpec((1, Q, NH * NP), lambda v: (v, 0, 0)),
            pl.BlockSpec((1, Q, NH * NP), lambda v: (v, 0, 0)),
        ],
        out_specs=[
            pl.BlockSpec((1, Q, 4 * NH * NP), lambda v: (v, 0, 0)),
            pl.BlockSpec((1, Q, 4 * NH * NP), lambda v: (v, 0, 0)),
        ],
        out_shape=[
            jax.ShapeDtypeStruct((V, Q, 4 * NH * NP), jnp.int32),
            jax.ShapeDtypeStruct((V, Q, 4 * NH * NP), jnp.float32),
        ],
    )(bev_query, swx, swy, sbx, sby, aw_w[...], aw_b.reshape(1, NH * NP),
      refx32, refy32)


def _ref_points(lidar2img):
    D = 4
    zs = jnp.linspace(0.5, 7.5, D, dtype=jnp.float32) / 8.0
    xs = jnp.linspace(0.5, 49.5, 50, dtype=jnp.float32) / 50.0
    ys = jnp.linspace(0.5, 49.5, 50, dtype=jnp.float32) / 50.0
    Xg = jnp.broadcast_to(xs[None, None, :], (D, 50, 50))
    Yg = jnp.broadcast_to(ys[None, :, None], (D, 50, 50))
    Zg = jnp.broadcast_to(zs[:, None, None], (D, 50, 50))
    ref3 = jnp.stack([Xg, Yg, Zg], -1).reshape(D, Q, 3)
    ref3 = jnp.broadcast_to(ref3[None], (1, D, Q, 3))
    rx = ref3[..., 0] * 102.4 + (-51.2)
    ry = ref3[..., 1] * 102.4 + (-51.2)
    rz = ref3[..., 2] * 8.0 + (-5.0)
    refh = jnp.stack([rx, ry, rz, jnp.ones_like(rx)], -1)
    refh = jnp.transpose(refh, (1, 0, 2, 3))
    cam = jnp.einsum('bvij,dbqj->dbvqi', lidar2img, refh)
    depth = cam[..., 2]
    bmask = depth > 1e-05
    uv = cam[..., 0:2] / jnp.maximum(depth[..., None], 1e-05)
    uv = jnp.transpose(uv, (2, 1, 3, 0, 4))
    bmask = jnp.transpose(bmask, (2, 1, 3, 0))
    u = uv[..., 0] * (1.0 / 16.0)
    v = uv[..., 1] * (1.0 / 16.0)
    in_img = (u >= 0) & (u <= WP - 1.0) & (v >= 0) & (v <= HP - 1.0)
    valid = bmask & in_img
    refx = jnp.clip(u / (WP - 1.0), 0.0, 1.0)
    refy = jnp.clip(v / (HP - 1.0), 0.0, 1.0)
    w = jnp.maximum(valid.astype(jnp.float32), EPS)
    wden = jnp.maximum(w.sum(-1), EPS)
    refx = (refx * w).sum(-1) / wden
    refy = (refy * w).sum(-1) / wden
    return refx[:, 0], refy[:, 0]


def _sc_gather(val48, idx48, w48):
    mesh = plsc.VectorSubcoreMesh(core_axis_name="c", subcore_axis_name="s")

    @functools.partial(
        pl.kernel, mesh=mesh,
        out_type=jax.ShapeDtypeStruct((V, NH, QPAD * HD), jnp.float32),
        scratch_types=[
            pltpu.VMEM((N * HD,), jnp.float32),
            pltpu.VMEM((QH * 16,), jnp.int32),
            pltpu.VMEM((QH * 16,), jnp.float32),
            pltpu.VMEM((QH * HD,), jnp.float32),
        ],
    )
    def k(val_hbm, idx_hbm, w_hbm, out_hbm, map_v, idx_v, w_v, out_v):
        wid = lax.axis_index("s") * 2 + lax.axis_index("c")
        for j in range(3):
            hw = wid * 3 + j
            t = hw // 2
            half = hw % 2
            vv = t // NH
            hh = t % NH
            qi0 = pl.multiple_of(half * (QH * 16), 128)
            qo0 = pl.multiple_of(half * (QH * HD), 128)
            pltpu.sync_copy(val_hbm.at[t], map_v)
            pltpu.sync_copy(idx_hbm.at[t, pl.ds(qi0, QH * 16)], idx_v)
            pltpu.sync_copy(w_hbm.at[t, pl.ds(qi0, QH * 16)], w_v)

            def qbody(qi, carry):
                acc0 = jnp.zeros((16,), jnp.float32)
                acc1 = jnp.zeros((16,), jnp.float32)
                iv = idx_v[pl.ds(qi * 16, 16)]
                wv_ = w_v[pl.ds(qi * 16, 16)]
                for kk in range(16):
                    n = iv[kk]
                    wk = wv_[kk]
                    acc0 = acc0 + wk * map_v[pl.ds(n * HD, 16)]
                    acc1 = acc1 + wk * map_v[pl.ds(n * HD + 16, 16)]
                out_v[pl.ds(qi * HD, 16)] = acc0
                out_v[pl.ds(qi * HD + 16, 16)] = acc1
                return carry

            lax.fori_loop(0, QH, qbody, 0)
            pltpu.sync_copy(out_v, out_hbm.at[vv, hh, pl.ds(qo0, QH * HD)])

    return k(val48, idx48, w48)


def _out_body(x_ref, wv_ref, opw_ref, opb_ref, qT_ref, out_ref, s_ref, ws_ref):
    v = pl.program_id(0)
    x = wv_ref[v]
    wv = jnp.maximum(x, 0.0) + jnp.log1p(jnp.exp(-jnp.abs(x)))

    @pl.when(v == 0)
    def _():
        s_ref[...] = jnp.zeros_like(s_ref)
        ws_ref[0] = 0.0

    xq = jnp.concatenate([x_ref[0, h, :Q] for h in range(NH)], axis=1)
    s_ref[...] += wv * xq
    ws_ref[0] += wv

    @pl.when(v == V - 1)
    def _():
        inv = 1.0 / jnp.maximum(ws_ref[0], EPS)
        sm = s_ref[...] * inv
        o = _dot(opw_ref[...], sm, ((1,), (1,)))
        out_ref[...] = o + opb_ref[...] + qT_ref[...]


def _fuse_out(deform, w_view, op_w, op_b, bev_query_t):
    return pl.pallas_call(
        _out_body,
        grid=(V,),
        in_specs=[
            pl.BlockSpec((1, NH, QPAD, HD), lambda v: (v, 0, 0, 0)),
            pl.BlockSpec(memory_space=pltpu.SMEM),
            pl.BlockSpec((C, C), lambda v: (0, 0)),
            pl.BlockSpec((C, 1), lambda v: (0, 0)),
            pl.BlockSpec((C, Q), lambda v: (0, 0)),
        ],
        out_specs=pl.BlockSpec((C, Q), lambda v: (0, 0)),
        out_shape=jax.ShapeDtypeStruct((C, Q), jnp.float32),
        scratch_shapes=[
            pltpu.VMEM((Q, C), jnp.float32),
            pltpu.SMEM((1,), jnp.float32),
        ],
    )(deform, w_view.reshape(V), op_w, op_b.reshape(C, 1), bev_query_t)


def kernel(last_tokens, lidar2img, Hp, Wp, w_view, proj_w, bev_query,
           so_w, so_b, aw_w, aw_b, vp_w, vp_b, op_w, op_b):
    tokens = last_tokens.reshape(V, N, CD)
    val48 = _value_maps(tokens, proj_w, vp_w, vp_b)
    refx, refy = _ref_points(lidar2img)
    refx32 = jnp.broadcast_to(refx[:, :, None], (V, Q, NH * NP))
    refy32 = jnp.broadcast_to(refy[:, :, None], (V, Q, NH * NP))
    idx6, w6 = _indices_weights(bev_query, so_w, so_b, aw_w, aw_b,
                                refx32, refy32)
    idx48 = jnp.transpose(idx6.reshape(V, Q, 4, NH, NP),
                          (0, 3, 1, 4, 2)).reshape(V * NH, Q, 16)
    w48 = jnp.transpose(w6.reshape(V, Q, 4, NH, NP),
                        (0, 3, 1, 4, 2)).reshape(V * NH, Q, 16)
    pad = ((0, 0), (0, QPAD - Q), (0, 0))
    idx48 = jnp.pad(idx48, pad).reshape(V * NH, QPAD * 16)
    w48 = jnp.pad(w48, pad).reshape(V * NH, QPAD * 16)
    deform = _sc_gather(val48.reshape(V * NH, N * HD), idx48, w48)
    deform = deform.reshape(V, NH, QPAD, HD)
    out = _fuse_out(deform, w_view, op_w, op_b, bev_query.T)
    return out.reshape(1, C, 50, 50)
